# baseline (speedup 1.0000x reference)
import jax
import jax.numpy as jnp
from jax import lax
from jax.experimental import pallas as pl
from jax.experimental.pallas import tpu as pltpu
from jax.experimental.pallas import tpu_sc as plsc

NUM_FIELDS = 26
VOCAB = 100000
EMBED_DIM = 16
BATCH = 16384
NUM_PAIRS = (NUM_FIELDS * (NUM_FIELDS - 1)) // 2

_INFO = plsc.get_sparse_core_info()
NC = _INFO.num_cores
NS = _INFO.num_subcores
NW = NC * NS
LANES = _INFO.num_lanes

CHUNK = 128
PER_W = BATCH // NW
N_ITERS = PER_W // CHUNK
N_SPANS = CHUNK * NUM_FIELDS // LANES


def _fam_body(x_hbm, w_hbm, out_hbm, xb, eb, ob, sem):
    wid = lax.axis_index("s") * NC + lax.axis_index("c")
    iota = lax.iota(jnp.int32, LANES)

    def chunk_body(t, _):
        base = wid * PER_W + t * CHUNK

        pltpu.sync_copy(
            x_hbm.at[pl.ds(base * NUM_FIELDS, CHUNK * NUM_FIELDS)], xb)

        def span_body(sp, offv):
            v = xb[pl.ds(sp * LANES, LANES)]
            xb[pl.ds(sp * LANES, LANES)] = v + offv
            nxt = offv + LANES * VOCAB
            return jnp.where(nxt >= NUM_FIELDS * VOCAB,
                             nxt - NUM_FIELDS * VOCAB, nxt)

        lax.fori_loop(0, N_SPANS, span_body, iota * VOCAB)

        pltpu.async_copy(w_hbm.at[xb], eb, sem).wait()

        mask15 = iota == (LANES - 1)

        def sample_body(s, _):
            e = [eb[s * NUM_FIELDS + f] for f in range(NUM_FIELDS)]
            obase_v = jnp.full((LANES,), 1, jnp.int32) * (s * NUM_PAIRS)
            p = 0
            for i in range(NUM_FIELDS - 1):
                for j in range(i + 1, NUM_FIELDS):
                    pv = plsc.cumsum(e[i] * e[j])
                    plsc.store_scatter(ob, [obase_v + p], pv, mask=mask15)
                    p += 1
            return 0

        lax.fori_loop(0, CHUNK, sample_body, 0)

        pltpu.sync_copy(ob, out_hbm.at[pl.ds(base * NUM_PAIRS,
                                             CHUNK * NUM_PAIRS)])
        return 0

    lax.fori_loop(0, N_ITERS, chunk_body, 0)


@jax.jit
def _fam(x_flat, w_flat):
    mesh = plsc.VectorSubcoreMesh(core_axis_name="c", subcore_axis_name="s")
    return pl.kernel(
        _fam_body,
        out_type=jax.ShapeDtypeStruct((BATCH * NUM_PAIRS,), jnp.float32),
        mesh=mesh,
        compiler_params=pltpu.CompilerParams(
            needs_layout_passes=False, use_tc_tiling_on_sc=False),
        scratch_types=[
            pltpu.VMEM((CHUNK * NUM_FIELDS,), jnp.int32),
            pltpu.VMEM((CHUNK * NUM_FIELDS, EMBED_DIM), jnp.float32),
            pltpu.VMEM((CHUNK * NUM_PAIRS,), jnp.float32),
            pltpu.SemaphoreType.DMA,
        ],
    )(x_flat, w_flat)


def kernel(x, W):
    x_flat = x.astype(jnp.int32).reshape(-1)
    w_flat = W.reshape(NUM_FIELDS * VOCAB, EMBED_DIM)
    return _fam(x_flat, w_flat).reshape(BATCH, NUM_PAIRS)
